# baseline (speedup 1.0000x reference)
import functools

import jax
import jax.numpy as jnp
from jax import lax
from jax.experimental import pallas as pl
from jax.experimental.pallas import tpu as pltpu
from jax.experimental.pallas import tpu_sc as plsc

_D = 64
_B = 16384 * 50
_BP = _B // 2
_NC = 2
_NS = 16
_NW = _NC * _NS
_PPW = _BP // _NW
_CHP = 128
_NCH = _PPW // _CHP
_NBUF = 4
_NGRP = _NCH // _NBUF


def _make_gather():
    mesh = plsc.VectorSubcoreMesh(core_axis_name="c", subcore_axis_name="s")

    @functools.partial(
        pl.kernel,
        mesh=mesh,
        out_type=jax.ShapeDtypeStruct((_BP, 2 * _D), jnp.float32),
        scratch_types=[
            pltpu.VMEM((_PPW,), jnp.int32),
            pltpu.VMEM((_PPW,), jnp.int32),
            pltpu.VMEM((_NBUF * _CHP, _D), jnp.float32),
            pltpu.VMEM((_NBUF * _CHP, _D), jnp.float32),
            pltpu.SemaphoreType.DMA((_NBUF,)),
            pltpu.SemaphoreType.DMA((_NBUF,)),
        ],
        compiler_params=pltpu.CompilerParams(use_tc_tiling_on_sc=False),
    )
    def gather_kernel(idx_ev_hbm, idx_od_hbm, table_hbm, out_hbm,
                      idx_ev_v, idx_od_v, rows_ev, rows_od, gsem, ssem):
        wid = lax.axis_index("s") * _NC + lax.axis_index("c")
        base = wid * _PPW
        pltpu.sync_copy(idx_ev_hbm.at[pl.ds(base, _PPW)], idx_ev_v)
        pltpu.sync_copy(idx_od_hbm.at[pl.ds(base, _PPW)], idx_od_v)

        def gather_pair(i, b, start):
            r0 = b * _CHP
            ev = pltpu.make_async_copy(
                table_hbm.at[idx_ev_v.at[pl.ds(i * _CHP, _CHP)]],
                rows_ev.at[pl.ds(r0, _CHP)],
                gsem.at[b],
            )
            od = pltpu.make_async_copy(
                table_hbm.at[idx_od_v.at[pl.ds(i * _CHP, _CHP)]],
                rows_od.at[pl.ds(r0, _CHP)],
                gsem.at[b],
            )
            if start:
                ev.start()
                od.start()
            else:
                ev.wait()
                od.wait()

        def store(i, b, start):
            r0 = b * _CHP
            ev = pltpu.make_async_copy(
                rows_ev.at[pl.ds(r0, _CHP)],
                out_hbm.at[pl.ds(base + i * _CHP, _CHP), pl.ds(0, _D)],
                ssem.at[b],
            )
            od = pltpu.make_async_copy(
                rows_od.at[pl.ds(r0, _CHP)],
                out_hbm.at[pl.ds(base + i * _CHP, _CHP), pl.ds(_D, _D)],
                ssem.at[b],
            )
            if start:
                ev.start()
                od.start()
            else:
                ev.wait()
                od.wait()

        for b in range(_NBUF):
            gather_pair(b, b, True)

        def body(g, carry):
            i0 = g * _NBUF
            for b in range(_NBUF):
                gather_pair(i0 + b, b, False)
                store(i0 + b, b, True)
            @pl.when(g < _NGRP - 1)
            def _():
                for b in range(_NBUF):
                    store(i0 + b, b, False)
                    gather_pair(i0 + _NBUF + b, b, True)

            return carry

        lax.fori_loop(0, _NGRP, body, 0)

        i0 = (_NGRP - 1) * _NBUF
        for b in range(_NBUF):
            store(i0 + b, b, False)

    return gather_kernel


_gather = _make_gather()


def kernel(token_ids, embedding):
    idx2 = token_ids.reshape(-1, 2)
    out = _gather(idx2[:, 0], idx2[:, 1], embedding)
    return out.reshape(*token_ids.shape, embedding.shape[1])


# speedup vs baseline: 1.3909x; 1.3909x over previous
import functools

import jax
import jax.numpy as jnp
from jax import lax
from jax.experimental import pallas as pl
from jax.experimental.pallas import tpu as pltpu
from jax.experimental.pallas import tpu_sc as plsc

_D = 64
_B = 16384 * 50
_NC = 2
_NS = 16
_NW = _NC * _NS
_RPW = _B // _NW
_CH = 128
_NCH = _RPW // _CH
_NBUF = 8
_NGRP = _NCH // _NBUF


def _make_gather():
    mesh = plsc.VectorSubcoreMesh(core_axis_name="c", subcore_axis_name="s")

    @functools.partial(
        pl.kernel,
        mesh=mesh,
        out_type=jax.ShapeDtypeStruct((_B, _D), jnp.float32),
        scratch_types=[
            pltpu.VMEM((_RPW,), jnp.int32),
            pltpu.VMEM((_NBUF * _CH, _D), jnp.float32),
            pltpu.SemaphoreType.DMA((_NBUF,)),
            pltpu.SemaphoreType.DMA((_NBUF,)),
        ],
        compiler_params=pltpu.CompilerParams(use_tc_tiling_on_sc=False),
    )
    def gather_kernel(idx_hbm, table_hbm, out_hbm, idx_v, rows_v, gsem, ssem):
        wid = lax.axis_index("s") * _NC + lax.axis_index("c")
        base = wid * _RPW
        pltpu.sync_copy(idx_hbm.at[pl.ds(base, _RPW)], idx_v)

        def gather(i, b, start):
            cp = pltpu.make_async_copy(
                table_hbm.at[idx_v.at[pl.ds(i * _CH, _CH)]],
                rows_v.at[pl.ds(b * _CH, _CH)],
                gsem.at[b],
            )
            cp.start() if start else cp.wait()

        def store(i, b, start):
            cp = pltpu.make_async_copy(
                rows_v.at[pl.ds(b * _CH, _CH)],
                out_hbm.at[pl.ds(base + i * _CH, _CH)],
                ssem.at[b],
            )
            cp.start() if start else cp.wait()

        for b in range(_NBUF):
            gather(b, b, True)

        def body(g, carry):
            i0 = g * _NBUF
            for b in range(_NBUF):
                gather(i0 + b, b, False)
                store(i0 + b, b, True)
            @pl.when(g < _NGRP - 1)
            def _():
                for b in range(_NBUF):
                    store(i0 + b, b, False)
                    gather(i0 + _NBUF + b, b, True)

            return carry

        lax.fori_loop(0, _NGRP, body, 0)

        i0 = (_NGRP - 1) * _NBUF
        for b in range(_NBUF):
            store(i0 + b, b, False)

    return gather_kernel


_gather = _make_gather()


def kernel(token_ids, embedding):
    idx = token_ids.reshape(-1)
    emb_lin = lax.optimization_barrier(embedding.reshape(-1))
    emb_rm = emb_lin.reshape(embedding.shape)
    out = _gather(idx, emb_rm)
    return out.reshape(*token_ids.shape, embedding.shape[1])


# speedup vs baseline: 2.0529x; 1.4760x over previous
import functools

import jax
import jax.numpy as jnp
from jax import lax
from jax.experimental import pallas as pl
from jax.experimental.pallas import tpu as pltpu
from jax.experimental.pallas import tpu_sc as plsc

_S = 50
_BT = 16384
_D = 64
_V = 100000
_NC = 2
_NS = 16
_NW = _NC * _NS
_DPW = _D // _NW
_BC = 4096
_NBC = _BT // _BC
_NCHUNK = _S * _NBC
_L = 16
_UNROLL = 8


def _make_gather():
    mesh = plsc.VectorSubcoreMesh(core_axis_name="c", subcore_axis_name="s")

    @functools.partial(
        pl.kernel,
        mesh=mesh,
        out_type=jax.ShapeDtypeStruct((_S, _D, _BT), jnp.float32),
        scratch_types=[
            pltpu.VMEM((_V,), jnp.float32),
            pltpu.VMEM((_BC,), jnp.int32),
            pltpu.VMEM((_BC,), jnp.int32),
            pltpu.VMEM((_BC,), jnp.float32),
            pltpu.VMEM((_BC,), jnp.float32),
            pltpu.SemaphoreType.DMA((2,)),
            pltpu.SemaphoreType.DMA((2,)),
        ],
        compiler_params=pltpu.CompilerParams(
            use_tc_tiling_on_sc=True, needs_layout_passes=False
        ),
    )
    def gather_kernel(tid_hbm, tab_hbm, out_hbm,
                      row_v, idx0, idx1, outb0, outb1, isem, osem):
        wid = lax.axis_index("s") * _NC + lax.axis_index("c")
        idx_bufs = (idx0, idx1)
        out_bufs = (outb0, outb1)

        def idx_copy(c, p):
            s, bc = c // _NBC, c % _NBC
            return pltpu.make_async_copy(
                tid_hbm.at[s, pl.ds(bc * _BC, _BC)], idx_bufs[p], isem.at[p]
            )

        def out_copy(d, c, p):
            s, bc = c // _NBC, c % _NBC
            return pltpu.make_async_copy(
                out_bufs[p], out_hbm.at[s, d, pl.ds(bc * _BC, _BC)], osem.at[p]
            )

        for dd in range(_DPW):
            d = wid * _DPW + dd
            pltpu.sync_copy(tab_hbm.at[d], row_v)
            idx_copy(0, 0).start()
            idx_copy(1, 1).start()

            def chunk(c, p, first, last):
                ib, ob = idx_bufs[p], out_bufs[p]
                idx_copy(c, p).wait()

                @pl.when(jnp.logical_not(first))
                def _():
                    out_copy(d, c - 2, p).wait()

                def gather_body(j, carry):
                    e0 = j * (_UNROLL * _L)
                    for u in range(_UNROLL):
                        e = e0 + u * _L
                        iv = ib[pl.ds(e, _L)]
                        ob[pl.ds(e, _L)] = plsc.load_gather(row_v, [iv])
                    return carry

                lax.fori_loop(0, _BC // (_UNROLL * _L), gather_body, 0)
                out_copy(d, c, p).start()

                @pl.when(jnp.logical_not(last))
                def _():
                    idx_copy(c + 2, p).start()

            def group(g, carry):
                first = (dd == 0) & (g == 0)
                chunk(2 * g, 0, first, 2 * g == _NCHUNK - 2)
                chunk(2 * g + 1, 1, first, 2 * g + 1 == _NCHUNK - 1)
                return carry

            lax.fori_loop(0, _NCHUNK // 2, group, 0)

        d_last = wid * _DPW + _DPW - 1
        out_copy(d_last, _NCHUNK - 2, 0).wait()
        out_copy(d_last, _NCHUNK - 1, 1).wait()

    return gather_kernel


_gather = _make_gather()


def kernel(token_ids, embedding):
    out_t = _gather(token_ids.T, embedding.T)
    return out_t.transpose(2, 0, 1)


# speedup vs baseline: 2.0665x; 1.0066x over previous
import functools

import jax
import jax.numpy as jnp
from jax import lax
from jax.experimental import pallas as pl
from jax.experimental.pallas import tpu as pltpu
from jax.experimental.pallas import tpu_sc as plsc

_S = 50
_BT = 16384
_D = 64
_V = 100000
_NC = 2
_NS = 16
_NW = _NC * _NS
_DPW = _D // _NW
_BC = 4096
_NBC = _BT // _BC
_NCHUNK = _S * _NBC
_L = 16
_UNROLL = 16


def _make_gather():
    mesh = plsc.VectorSubcoreMesh(core_axis_name="c", subcore_axis_name="s")

    @functools.partial(
        pl.kernel,
        mesh=mesh,
        out_type=jax.ShapeDtypeStruct((_S, _D, _BT), jnp.float32),
        scratch_types=[
            pltpu.VMEM((_V,), jnp.float32),
            pltpu.VMEM((_BC,), jnp.int32),
            pltpu.VMEM((_BC,), jnp.int32),
            pltpu.VMEM((_BC,), jnp.float32),
            pltpu.VMEM((_BC,), jnp.float32),
            pltpu.SemaphoreType.DMA((2,)),
            pltpu.SemaphoreType.DMA((2,)),
        ],
        compiler_params=pltpu.CompilerParams(
            use_tc_tiling_on_sc=True, needs_layout_passes=False
        ),
    )
    def gather_kernel(tid_hbm, tab_hbm, out_hbm,
                      row_v, idx0, idx1, outb0, outb1, isem, osem):
        wid = lax.axis_index("s") * _NC + lax.axis_index("c")
        idx_bufs = (idx0, idx1)
        out_bufs = (outb0, outb1)

        def idx_copy(c, p):
            s, bc = c // _NBC, c % _NBC
            return pltpu.make_async_copy(
                tid_hbm.at[s, pl.ds(bc * _BC, _BC)], idx_bufs[p], isem.at[p]
            )

        def out_copy(d, c, p):
            s, bc = c // _NBC, c % _NBC
            return pltpu.make_async_copy(
                out_bufs[p], out_hbm.at[s, d, pl.ds(bc * _BC, _BC)], osem.at[p]
            )

        for dd in range(_DPW):
            d = wid * _DPW + dd
            pltpu.sync_copy(tab_hbm.at[d], row_v)
            idx_copy(0, 0).start()
            idx_copy(1, 1).start()

            def chunk(c, p, first, last):
                ib, ob = idx_bufs[p], out_bufs[p]
                idx_copy(c, p).wait()

                @pl.when(jnp.logical_not(first))
                def _():
                    out_copy(d, c - 2, p).wait()

                def gather_body(j, carry):
                    e0 = j * (_UNROLL * _L)
                    for u in range(_UNROLL):
                        e = e0 + u * _L
                        iv = ib[pl.ds(e, _L)]
                        ob[pl.ds(e, _L)] = plsc.load_gather(row_v, [iv])
                    return carry

                lax.fori_loop(0, _BC // (_UNROLL * _L), gather_body, 0)
                out_copy(d, c, p).start()

                @pl.when(jnp.logical_not(last))
                def _():
                    idx_copy(c + 2, p).start()

            def group(g, carry):
                first = (dd == 0) & (g == 0)
                chunk(2 * g, 0, first, 2 * g == _NCHUNK - 2)
                chunk(2 * g + 1, 1, first, 2 * g + 1 == _NCHUNK - 1)
                return carry

            lax.fori_loop(0, _NCHUNK // 2, group, 0)

        d_last = wid * _DPW + _DPW - 1
        out_copy(d_last, _NCHUNK - 2, 0).wait()
        out_copy(d_last, _NCHUNK - 1, 1).wait()

    return gather_kernel


_gather = _make_gather()


def kernel(token_ids, embedding):
    out_t = _gather(token_ids.T, embedding.T)
    return out_t.transpose(2, 0, 1)


# speedup vs baseline: 3.0251x; 1.4639x over previous
import functools

import jax
import jax.numpy as jnp
from jax import lax
from jax.experimental import pallas as pl
from jax.experimental.pallas import tpu as pltpu
from jax.experimental.pallas import tpu_sc as plsc

_S = 50
_BT = 16384
_D = 64
_V = 100000
_NC = 2
_NS = 16
_NW = _NC * _NS
_DPW = _D // _NW
_BC = 4096
_NBC = _BT // _BC
_NCHUNK = _S * _NBC
_L = 16
_UNROLL = 16


def _make_gather():
    mesh = plsc.VectorSubcoreMesh(core_axis_name="c", subcore_axis_name="s")

    @functools.partial(
        pl.kernel,
        mesh=mesh,
        out_type=jax.ShapeDtypeStruct((_S, _D, _BT), jnp.float32),
        scratch_types=[
            pltpu.VMEM((_V,), jnp.float32),
            pltpu.VMEM((_BC,), jnp.int32),
            pltpu.VMEM((_BC,), jnp.int32),
            pltpu.VMEM((_BC,), jnp.float32),
            pltpu.VMEM((_BC,), jnp.float32),
            pltpu.SemaphoreType.DMA((2,)),
            pltpu.SemaphoreType.DMA((2,)),
        ],
        compiler_params=pltpu.CompilerParams(
            use_tc_tiling_on_sc=True, needs_layout_passes=False
        ),
    )
    def gather_kernel(tid_hbm, tab_hbm, out_hbm,
                      row_v, idx0, idx1, outb0, outb1, isem, osem):
        wid = lax.axis_index("s") * _NC + lax.axis_index("c")
        idx_bufs = (idx0, idx1)
        out_bufs = (outb0, outb1)

        def idx_copy(c, p):
            s, bc = c // _NBC, c % _NBC
            return pltpu.make_async_copy(
                tid_hbm.at[s, pl.ds(bc * _BC, _BC)], idx_bufs[p], isem.at[p]
            )

        def out_copy(d, c, p):
            s, bc = c // _NBC, c % _NBC
            return pltpu.make_async_copy(
                out_bufs[p], out_hbm.at[s, d, pl.ds(bc * _BC, _BC)], osem.at[p]
            )

        for dd in range(_DPW):
            d = wid * _DPW + dd
            pltpu.sync_copy(tab_hbm.at[d], row_v)
            idx_copy(0, 0).start()
            idx_copy(1, 1).start()

            def chunk(c, p, first, last):
                ib, ob = idx_bufs[p], out_bufs[p]
                idx_copy(c, p).wait()

                @pl.when(jnp.logical_not(first))
                def _():
                    out_copy(d, c - 2, p).wait()

                @plsc.parallel_loop(0, _BC // _L, unroll=_UNROLL)
                def gather_body(j):
                    e = j * _L
                    iv = ib[pl.ds(e, _L)]
                    ob[pl.ds(e, _L)] = plsc.load_gather(row_v, [iv])
                out_copy(d, c, p).start()

                @pl.when(jnp.logical_not(last))
                def _():
                    idx_copy(c + 2, p).start()

            def group(g, carry):
                first = (dd == 0) & (g == 0)
                chunk(2 * g, 0, first, 2 * g == _NCHUNK - 2)
                chunk(2 * g + 1, 1, first, 2 * g + 1 == _NCHUNK - 1)
                return carry

            lax.fori_loop(0, _NCHUNK // 2, group, 0)

        d_last = wid * _DPW + _DPW - 1
        out_copy(d_last, _NCHUNK - 2, 0).wait()
        out_copy(d_last, _NCHUNK - 1, 1).wait()

    return gather_kernel


_gather = _make_gather()


def kernel(token_ids, embedding):
    out_t = _gather(token_ids.T, embedding.T)
    return out_t.transpose(2, 0, 1)


# speedup vs baseline: 3.0255x; 1.0001x over previous
import functools

import jax
import jax.numpy as jnp
from jax import lax
from jax.experimental import pallas as pl
from jax.experimental.pallas import tpu as pltpu
from jax.experimental.pallas import tpu_sc as plsc

_S = 50
_BT = 16384
_D = 64
_V = 100000
_NC = 2
_NS = 16
_NW = _NC * _NS
_DPW = _D // _NW
_BC = 4096
_NBC = _BT // _BC
_NCHUNK = _S * _NBC
_L = 16
_UNROLL = 32


def _make_gather():
    mesh = plsc.VectorSubcoreMesh(core_axis_name="c", subcore_axis_name="s")

    @functools.partial(
        pl.kernel,
        mesh=mesh,
        out_type=jax.ShapeDtypeStruct((_S, _D, _BT), jnp.float32),
        scratch_types=[
            pltpu.VMEM((_V,), jnp.float32),
            pltpu.VMEM((_BC,), jnp.int32),
            pltpu.VMEM((_BC,), jnp.int32),
            pltpu.VMEM((_BC,), jnp.float32),
            pltpu.VMEM((_BC,), jnp.float32),
            pltpu.SemaphoreType.DMA((2,)),
            pltpu.SemaphoreType.DMA((2,)),
        ],
        compiler_params=pltpu.CompilerParams(
            use_tc_tiling_on_sc=True, needs_layout_passes=False
        ),
    )
    def gather_kernel(tid_hbm, tab_hbm, out_hbm,
                      row_v, idx0, idx1, outb0, outb1, isem, osem):
        wid = lax.axis_index("s") * _NC + lax.axis_index("c")
        idx_bufs = (idx0, idx1)
        out_bufs = (outb0, outb1)

        def idx_copy(c, p):
            s, bc = c // _NBC, c % _NBC
            return pltpu.make_async_copy(
                tid_hbm.at[s, pl.ds(bc * _BC, _BC)], idx_bufs[p], isem.at[p]
            )

        def out_copy(d, c, p):
            s, bc = c // _NBC, c % _NBC
            return pltpu.make_async_copy(
                out_bufs[p], out_hbm.at[s, d, pl.ds(bc * _BC, _BC)], osem.at[p]
            )

        for dd in range(_DPW):
            d = wid * _DPW + dd
            pltpu.sync_copy(tab_hbm.at[d], row_v)
            idx_copy(0, 0).start()
            idx_copy(1, 1).start()

            def chunk(c, p, first, last):
                ib, ob = idx_bufs[p], out_bufs[p]
                idx_copy(c, p).wait()

                @pl.when(jnp.logical_not(first))
                def _():
                    out_copy(d, c - 2, p).wait()

                @plsc.parallel_loop(0, _BC // _L, unroll=_UNROLL)
                def gather_body(j):
                    e = j * _L
                    iv = ib[pl.ds(e, _L)]
                    ob[pl.ds(e, _L)] = plsc.load_gather(row_v, [iv])
                out_copy(d, c, p).start()

                @pl.when(jnp.logical_not(last))
                def _():
                    idx_copy(c + 2, p).start()

            def group(g, carry):
                first = (dd == 0) & (g == 0)
                chunk(2 * g, 0, first, 2 * g == _NCHUNK - 2)
                chunk(2 * g + 1, 1, first, 2 * g + 1 == _NCHUNK - 1)
                return carry

            lax.fori_loop(0, _NCHUNK // 2, group, 0)

        d_last = wid * _DPW + _DPW - 1
        out_copy(d_last, _NCHUNK - 2, 0).wait()
        out_copy(d_last, _NCHUNK - 1, 1).wait()

    return gather_kernel


_gather = _make_gather()


def kernel(token_ids, embedding):
    out_t = _gather(token_ids.T, embedding.T)
    return out_t.transpose(2, 0, 1)
